# baseline (speedup 1.0000x reference)
import jax
import jax.numpy as jnp
from jax.experimental import pallas as pl


def _add_body(x_ref, e_ref, o_ref):
    o_ref[...] = x_ref[...] + e_ref[...]


def kernel(x, emb_weight):
    seq_len, dim = x.shape
    block_rows = 2048
    grid = (seq_len // block_rows,)
    spec = pl.BlockSpec((block_rows, dim), lambda i: (i, 0))
    return pl.pallas_call(
        _add_body,
        grid=grid,
        in_specs=[spec, spec],
        out_specs=spec,
        out_shape=jax.ShapeDtypeStruct((seq_len, dim), x.dtype),
    )(x, emb_weight[:seq_len])
